# Initial kernel scaffold; baseline (speedup 1.0000x reference)
import jax
import jax.numpy as jnp
from jax.experimental import pallas as pl


def kernel(x, edge_index_rel0, edge_index_rel1, edge_index_rel2, W0_rel0, W0_rel1, W0_rel2, b0_rel0, b0_rel1, b0_rel2, W1_rel0, W1_rel1, W1_rel2, b1_rel0, b1_rel1, b1_rel2):
    raise NotImplementedError("write your pallas kernel here")


# speedup vs baseline: 1.2068x; 1.2068x over previous
import functools
import jax
import jax.numpy as jnp
from jax import lax
from jax.experimental import pallas as pl
from jax.experimental.pallas import tpu as pltpu, tpu_sc as plsc

N = 50000
E = 200000
D = 128
H = 128
C = 128
NCH = 391
NP = NCH * C
NF = 8 * NP
EPAD = ((E + C - 1) // C) * C
G = EPAD // C
NS = 16
F = 16
NBLK = H // F
BPC = NBLK // 2
DEGW = 8
NDCH = 25
DROWCH = N // NDCH

_mesh = plsc.VectorSubcoreMesh(core_axis_name="c", subcore_axis_name="s")
_sc_params = pltpu.CompilerParams(use_tc_tiling_on_sc=False)


@functools.partial(
    pl.kernel,
    out_type=[jax.ShapeDtypeStruct((N, DEGW), jnp.float32) for _ in range(6)],
    mesh=_mesh,
    scratch_types=[
        pltpu.VMEM((C,), jnp.int32),
        pltpu.VMEM((DROWCH, DEGW), jnp.float32),
        pltpu.VMEM((C, DEGW), jnp.float32),
        pltpu.VMEM_SHARED((NP, DEGW), jnp.float32),
        pltpu.SemaphoreType.DMA,
    ],
    compiler_params=_sc_params,
)
def _deg_kernel(src0, dst0, src1, dst1, src2, dst2, ones_h, zeros_h,
                o0, o1, o2, i0, i1, i2,
                idx_v, wb_v, ones_v, acc, sem):
    c = lax.axis_index("c")
    s = lax.axis_index("s")
    srcs = [src0, src1, src2]
    dsts = [dst0, dst1, dst2]
    outs_o = [o0, o1, o2]
    outs_i = [i0, i1, i2]

    pltpu.sync_copy(ones_h, ones_v)
    pltpu.sync_copy(zeros_h, wb_v)

    for side in range(2):
        @pl.when(c == side)
        def _():
            edges = dsts if side else srcs
            outs = outs_i if side else outs_o
            for r in range(3):
                def zero_chunk(i, _):
                    k = s + i * NS
                    @pl.when(k < NDCH)
                    def _():
                        pltpu.sync_copy(wb_v, acc.at[pl.ds(k * DROWCH, DROWCH)])
                    return 0
                lax.fori_loop(0, pl.cdiv(NDCH, NS), zero_chunk, 0)
                @pl.when(s == 0)
                def _():
                    pltpu.sync_copy(wb_v.at[pl.ds(0, NP - N)], acc.at[pl.ds(N, NP - N)])
                plsc.subcore_barrier()

                def edge_chunk(i, _):
                    j = s + i * NS
                    @pl.when(j < G)
                    def _():
                        pltpu.sync_copy(edges[r].at[pl.ds(j * C, C)], idx_v)
                        pltpu.sync_copy(ones_v, acc.at[idx_v], add=True)
                    return 0
                lax.fori_loop(0, pl.cdiv(G, NS), edge_chunk, 0)
                plsc.subcore_barrier()

                def wb_chunk(i, _):
                    k = s + i * NS
                    @pl.when(k < NDCH)
                    def _():
                        pltpu.sync_copy(acc.at[pl.ds(k * DROWCH, DROWCH)], wb_v)
                        pltpu.sync_copy(wb_v, outs[r].at[pl.ds(k * DROWCH, DROWCH)])
                        pltpu.sync_copy(zeros_h, wb_v)
                    return 0
                lax.fori_loop(0, pl.cdiv(NDCH, NS), wb_chunk, 0)
                plsc.subcore_barrier()


@functools.partial(
    pl.kernel,
    out_type=[jax.ShapeDtypeStruct((NF, F), jnp.float32) for _ in range(3)],
    mesh=_mesh,
    scratch_types=[
        pltpu.VMEM((C,), jnp.int32),
        pltpu.VMEM((C,), jnp.int32),
        pltpu.VMEM((C,), jnp.int32),
        pltpu.VMEM((C,), jnp.int32),
        pltpu.VMEM((C, F), jnp.float32),
        pltpu.VMEM((C, F), jnp.float32),
        pltpu.VMEM((C, F), jnp.float32),
        pltpu.VMEM_SHARED((NP, F), jnp.float32),
        pltpu.SemaphoreType.DMA,
    ],
    compiler_params=_sc_params,
)
def _agg_kernel(t0, t1, t2, src0, dst0, src1, dst1, src2, dst2,
                a0, a1, a2,
                src_v, dst_v, gi_v, wi_v, msg_v, wb_v, z_v, acc, sem):
    c = lax.axis_index("c")
    s = lax.axis_index("s")
    tables = [t0, t1, t2]
    srcs = [src0, src1, src2]
    dsts = [dst0, dst1, dst2]
    outs = [a0, a1, a2]
    iota = lax.iota(jnp.int32, 16)

    for i in range(C):
        z_v[i, :] = jnp.zeros((F,), jnp.float32)

    for r in range(3):
        for b in range(NBLK):
            @pl.when(c == b // BPC)
            def _():
                def zero_chunk(i, _):
                    m = s + i * NS
                    @pl.when(m < NCH)
                    def _():
                        pltpu.sync_copy(z_v, acc.at[pl.ds(m * C, C)])
                    return 0
                lax.fori_loop(0, pl.cdiv(NCH, NS), zero_chunk, 0)
                plsc.subcore_barrier()

                def edge_chunk(i, _):
                    j = s + i * NS
                    @pl.when(j < G)
                    def _():
                        pltpu.sync_copy(srcs[r].at[pl.ds(j * C, C)], src_v)
                        pltpu.sync_copy(dsts[r].at[pl.ds(j * C, C)], dst_v)
                        for g in range(C // 16):
                            gi_v[pl.ds(g * 16, 16)] = src_v[pl.ds(g * 16, 16)] * 8 + b
                        pltpu.async_copy(tables[r].at[gi_v], msg_v, sem).wait()
                        pltpu.sync_copy(msg_v, acc.at[dst_v], add=True)
                    return 0
                lax.fori_loop(0, pl.cdiv(G, NS), edge_chunk, 0)
                plsc.subcore_barrier()

                def wb_chunk(i, _):
                    m = s + i * NS
                    @pl.when(m < NCH)
                    def _():
                        pltpu.sync_copy(acc.at[pl.ds(m * C, C)], wb_v)
                        for g in range(C // 16):
                            wi_v[pl.ds(g * 16, 16)] = (m * C + g * 16 + iota) * 8 + b
                        pltpu.async_copy(wb_v, outs[r].at[wi_v], sem).wait()
                    return 0
                lax.fori_loop(0, pl.cdiv(NCH, NS), wb_chunk, 0)
                plsc.subcore_barrier()


BN = 1000
GN = N // BN
GP = (NP + BN - 1) // BN


def _clamped(i):
    return (jnp.minimum(i, GN - 1), 0)


def _csrc(deg_blk):
    return lax.rsqrt(jnp.maximum(deg_blk[:, :1], 1.0))


def _prep_body(x_ref, d0_ref, d1_ref, d2_ref, w0_ref, w1_ref, w2_ref,
               o0_ref, o1_ref, o2_ref):
    h = x_ref[...]
    degs = [d0_ref, d1_ref, d2_ref]
    ws = [w0_ref, w1_ref, w2_ref]
    for r, o_ref in enumerate([o0_ref, o1_ref, o2_ref]):
        o_ref[...] = jnp.dot(h * _csrc(degs[r][...]), ws[r][...],
                             preferred_element_type=jnp.float32)


_prep = pl.pallas_call(
    _prep_body,
    grid=(GP,),
    in_specs=[pl.BlockSpec((BN, D), _clamped)]
    + [pl.BlockSpec((BN, DEGW), _clamped) for _ in range(3)]
    + [pl.BlockSpec((D, H), lambda i: (0, 0)) for _ in range(3)],
    out_specs=[pl.BlockSpec((BN, H), lambda i: (i, 0)) for _ in range(3)],
    out_shape=[jax.ShapeDtypeStruct((NP, H), jnp.float32) for _ in range(3)],
)


def _relu_sum(agg, din, bs):
    acc = bs[0][...] + bs[1][...] + bs[2][...]
    acc = jnp.broadcast_to(acc, (BN, H)).astype(jnp.float32)
    for r in range(3):
        acc = acc + agg[r][...] * _csrc(din[r][...])
    return jnp.maximum(acc, 0.0)


def _mid_body(*refs):
    agg = refs[0:3]
    din = refs[3:6]
    dout = refs[6:9]
    ws = refs[9:12]
    bs = refs[12:15]
    out_refs = refs[15:18]
    h1 = _relu_sum(agg, din, bs)
    for r in range(3):
        out_refs[r][...] = jnp.dot(h1 * _csrc(dout[r][...]), ws[r][...],
                                   preferred_element_type=jnp.float32)


_mid = pl.pallas_call(
    _mid_body,
    grid=(GP,),
    in_specs=[pl.BlockSpec((BN, H), _clamped) for _ in range(3)]
    + [pl.BlockSpec((BN, DEGW), _clamped) for _ in range(6)]
    + [pl.BlockSpec((D, H), lambda i: (0, 0)) for _ in range(3)]
    + [pl.BlockSpec((1, H), lambda i: (0, 0)) for _ in range(3)],
    out_specs=[pl.BlockSpec((BN, H), lambda i: (i, 0)) for _ in range(3)],
    out_shape=[jax.ShapeDtypeStruct((NP, H), jnp.float32) for _ in range(3)],
)


def _post_body(*refs):
    agg = refs[0:3]
    din = refs[3:6]
    bs = refs[6:9]
    refs[9][...] = _relu_sum(agg, din, bs)


_post = pl.pallas_call(
    _post_body,
    grid=(GN,),
    in_specs=[pl.BlockSpec((BN, H), lambda i: (i, 0)) for _ in range(3)]
    + [pl.BlockSpec((BN, DEGW), lambda i: (i, 0)) for _ in range(3)]
    + [pl.BlockSpec((1, H), lambda i: (0, 0)) for _ in range(3)],
    out_specs=pl.BlockSpec((BN, H), lambda i: (i, 0)),
    out_shape=jax.ShapeDtypeStruct((N, H), jnp.float32),
)


@jax.jit
def kernel(x, edge_index_rel0, edge_index_rel1, edge_index_rel2,
           W0_rel0, W0_rel1, W0_rel2, b0_rel0, b0_rel1, b0_rel2,
           W1_rel0, W1_rel1, W1_rel2, b1_rel0, b1_rel1, b1_rel2):
    pad = jnp.full((EPAD - E,), N, dtype=jnp.int32)
    edges = []
    for ei in (edge_index_rel0, edge_index_rel1, edge_index_rel2):
        edges.append(jnp.concatenate([ei[0], pad]))
        edges.append(jnp.concatenate([ei[1], pad]))

    ones_h = jnp.ones((C, DEGW), jnp.float32)
    zeros_deg = jnp.zeros((DROWCH, DEGW), jnp.float32)

    degs = _deg_kernel(*edges, ones_h, zeros_deg)
    dego = degs[0:3]
    degi = degs[3:6]

    b0s = [b0_rel0.reshape(1, H), b0_rel1.reshape(1, H), b0_rel2.reshape(1, H)]
    b1s = [b1_rel0.reshape(1, H), b1_rel1.reshape(1, H), b1_rel2.reshape(1, H)]

    tables0 = _prep(x, *dego, W0_rel0, W0_rel1, W0_rel2)
    aggs0 = _agg_kernel(*[t.reshape(NF, F) for t in tables0], *edges)
    tables1 = _mid(*[a.reshape(NP, H) for a in aggs0], *degi, *dego,
                   W1_rel0, W1_rel1, W1_rel2, *b0s)
    aggs1 = _agg_kernel(*[t.reshape(NF, F) for t in tables1], *edges)
    return _post(*[a.reshape(NP, H) for a in aggs1], *degi, *b1s)


# speedup vs baseline: 2.8412x; 2.3543x over previous
import functools
import jax
import jax.numpy as jnp
from jax import lax
from jax.experimental import pallas as pl
from jax.experimental.pallas import tpu as pltpu, tpu_sc as plsc

N = 50000
E = 200000
D = 128
H = 128
C = 128
NS = 16
Q = 100
EPAD = NS * Q * C
PD = 4
TCH = 25
NP = NS * TCH * C
NF = 8 * NP
F = 16
NBLK = H // F
BPC = NBLK // 2
ZCH = 1024
DEGW = 8
DTCH = 8
DROWCH = NP // (NS * DTCH)

_mesh = plsc.VectorSubcoreMesh(core_axis_name="c", subcore_axis_name="s")
_sc_params = pltpu.CompilerParams(use_tc_tiling_on_sc=False)


@functools.partial(
    pl.kernel,
    out_type=[jax.ShapeDtypeStruct((N, DEGW), jnp.float32) for _ in range(6)],
    mesh=_mesh,
    scratch_types=[
        pltpu.VMEM((Q * C,), jnp.int32),
        pltpu.VMEM((C,), jnp.int32),
        pltpu.VMEM((C,), jnp.int32),
        pltpu.VMEM((DROWCH, DEGW), jnp.float32),
        pltpu.VMEM((DROWCH, DEGW), jnp.float32),
        pltpu.VMEM((C, DEGW), jnp.float32),
        pltpu.VMEM_SHARED((NP, DEGW), jnp.float32),
        pltpu.SemaphoreType.DMA,
        pltpu.SemaphoreType.DMA,
    ],
    compiler_params=_sc_params,
)
def _deg_kernel(src0, dst0, src1, dst1, src2, dst2, ones_h, zeros_h,
                o0, o1, o2, i0, i1, i2,
                idx_blk, ix0, ix1, zero_v, wbd_v, ones_v, acc, sem0, sem1):
    c = lax.axis_index("c")
    s = lax.axis_index("s")
    srcs = [src0, src1, src2]
    dsts = [dst0, dst1, dst2]
    outs_o = [o0, o1, o2]
    outs_i = [i0, i1, i2]
    ixs = [ix0, ix1]
    sems = [sem0, sem1]

    pltpu.sync_copy(ones_h, ones_v)
    pltpu.sync_copy(zeros_h, zero_v)

    for side in range(2):
        @pl.when(c == side)
        def _():
            edges = dsts if side else srcs
            outs = outs_i if side else outs_o
            for r in range(3):
                def zero_chunk(i, _):
                    pltpu.sync_copy(zero_v, acc.at[pl.ds((s * DTCH + i) * DROWCH, DROWCH)])
                    return 0
                lax.fori_loop(0, DTCH, zero_chunk, 0)
                plsc.subcore_barrier()

                pltpu.sync_copy(edges[r].at[pl.ds(s * Q * C, Q * C)], idx_blk)

                def fill_ix(j, k):
                    for g in range(C // 16):
                        ixs[k][pl.ds(g * 16, 16)] = idx_blk[pl.ds(j * C + g * 16, 16)]

                def fire(k):
                    pltpu.async_copy(ones_v, acc.at[ixs[k]], sems[k], add=True)

                for k in range(2):
                    fill_ix(k, k)
                    fire(k)

                def edge_pair(i, _):
                    j = i * 2
                    for k in range(2):
                        pltpu.make_async_copy(ones_v, acc.at[ixs[k]], sems[k]).wait()
                        jn = j + k + 2
                        @pl.when(jn < Q)
                        def _():
                            fill_ix(jn, k)
                            fire(k)
                    return 0
                lax.fori_loop(0, Q // 2, edge_pair, 0)
                plsc.subcore_barrier()

                def wb_chunk(i, _):
                    base = (s * DTCH + i) * DROWCH
                    @pl.when(base < N)
                    def _():
                        pltpu.sync_copy(acc.at[pl.ds(base, DROWCH)], wbd_v)
                        pltpu.sync_copy(wbd_v, outs[r].at[pl.ds(base, DROWCH)])
                    return 0
                lax.fori_loop(0, DTCH, wb_chunk, 0)
                plsc.subcore_barrier()


@functools.partial(
    pl.kernel,
    out_type=[jax.ShapeDtypeStruct((NF, F), jnp.float32) for _ in range(3)],
    mesh=_mesh,
    scratch_types=[
        pltpu.VMEM((Q * C,), jnp.int32),
        pltpu.VMEM((Q * C,), jnp.int32),
        [pltpu.VMEM((C,), jnp.int32) for _ in range(PD)],
        [pltpu.VMEM((C,), jnp.int32) for _ in range(PD)],
        [pltpu.VMEM((C, F), jnp.float32) for _ in range(PD)],
        [pltpu.VMEM((C,), jnp.int32) for _ in range(2)],
        [pltpu.VMEM((C, F), jnp.float32) for _ in range(2)],
        pltpu.VMEM((ZCH, F), jnp.float32),
        pltpu.VMEM_SHARED((NP, F), jnp.float32),
        [pltpu.SemaphoreType.DMA for _ in range(PD)],
        [pltpu.SemaphoreType.DMA for _ in range(2)],
    ],
    compiler_params=_sc_params,
)
def _agg_kernel(t0, t1, t2, src0, dst0, src1, dst1, src2, dst2,
                a0, a1, a2,
                src_blk, dst_blk, gi_v, di_v, msg_v, wi_v, wb_v, z_v, acc,
                gsem, wsem):
    c = lax.axis_index("c")
    s = lax.axis_index("s")
    tables = [t0, t1, t2]
    srcs = [src0, src1, src2]
    dsts = [dst0, dst1, dst2]
    outs = [a0, a1, a2]
    iota = lax.iota(jnp.int32, 16)

    def zz(i, _):
        z_v[i, :] = jnp.zeros((F,), jnp.float32)
        return 0
    lax.fori_loop(0, ZCH, zz, 0)

    base_e = s * Q * C
    base_n = s * TCH * C

    for r in range(3):
        pltpu.sync_copy(srcs[r].at[pl.ds(base_e, Q * C)], src_blk)
        pltpu.sync_copy(dsts[r].at[pl.ds(base_e, Q * C)], dst_blk)

        def block_body(b, _):
            @pl.when(c == b // BPC)
            def _():
                for q in range(TCH * C // ZCH):
                    pltpu.sync_copy(z_v, acc.at[pl.ds(base_n + q * ZCH, ZCH)])
                rem = TCH * C - (TCH * C // ZCH) * ZCH
                if rem:
                    pltpu.sync_copy(z_v.at[pl.ds(0, rem)],
                                    acc.at[pl.ds(base_n + (TCH * C // ZCH) * ZCH, rem)])
                plsc.subcore_barrier()

                def prep(j, k):
                    for g in range(C // 16):
                        sv = src_blk[pl.ds(j * C + g * 16, 16)]
                        gi_v[k][pl.ds(g * 16, 16)] = sv * 8 + b
                        di_v[k][pl.ds(g * 16, 16)] = dst_blk[pl.ds(j * C + g * 16, 16)]

                def fire(k):
                    pltpu.async_copy(tables[r].at[gi_v[k]], msg_v[k], gsem[k])

                for k in range(PD):
                    prep(k, k)
                    fire(k)

                def edge_group(i, _):
                    j = i * PD
                    for k in range(PD):
                        pltpu.make_async_copy(tables[r].at[gi_v[k]], msg_v[k], gsem[k]).wait()
                        pltpu.sync_copy(msg_v[k], acc.at[di_v[k]], add=True)
                        jn = j + k + PD
                        @pl.when(jn < Q)
                        def _():
                            prep(jn, k)
                            fire(k)
                    return 0
                lax.fori_loop(0, Q // PD, edge_group, 0)
                plsc.subcore_barrier()

                def wb_prep(m, k):
                    pltpu.sync_copy(acc.at[pl.ds(base_n + m * C, C)], wb_v[k])
                    for g in range(C // 16):
                        wi_v[k][pl.ds(g * 16, 16)] = (base_n + m * C + g * 16 + iota) * 8 + b

                def wb_fire(k):
                    pltpu.async_copy(wb_v[k], outs[r].at[wi_v[k]], wsem[k])

                def wb_wait(k):
                    pltpu.make_async_copy(wb_v[k], outs[r].at[wi_v[k]], wsem[k]).wait()

                for k in range(2):
                    wb_prep(k, k)
                    wb_fire(k)

                def wb_pair(i, _):
                    m = i * 2
                    for k in range(2):
                        wb_wait(k)
                        mn = m + k + 2
                        @pl.when(mn < TCH)
                        def _():
                            wb_prep(mn, k)
                            wb_fire(k)
                    return 0
                lax.fori_loop(0, TCH // 2, wb_pair, 0)
                wb_wait(0)
                plsc.subcore_barrier()
            return 0

        lax.fori_loop(0, NBLK, block_body, 0)


BN = 1000
GN = N // BN
GP = (NP + BN - 1) // BN


def _clamped(i):
    return (jnp.minimum(i, GN - 1), 0)


def _csrc(deg_blk):
    return lax.rsqrt(jnp.maximum(deg_blk[:, :1], 1.0))


def _prep_body(x_ref, d0_ref, d1_ref, d2_ref, w0_ref, w1_ref, w2_ref,
               o0_ref, o1_ref, o2_ref):
    h = x_ref[...]
    degs = [d0_ref, d1_ref, d2_ref]
    ws = [w0_ref, w1_ref, w2_ref]
    for r, o_ref in enumerate([o0_ref, o1_ref, o2_ref]):
        o_ref[...] = jnp.dot(h * _csrc(degs[r][...]), ws[r][...],
                             preferred_element_type=jnp.float32)


_prep = pl.pallas_call(
    _prep_body,
    grid=(GP,),
    in_specs=[pl.BlockSpec((BN, D), _clamped)]
    + [pl.BlockSpec((BN, DEGW), _clamped) for _ in range(3)]
    + [pl.BlockSpec((D, H), lambda i: (0, 0)) for _ in range(3)],
    out_specs=[pl.BlockSpec((BN, H), lambda i: (i, 0)) for _ in range(3)],
    out_shape=[jax.ShapeDtypeStruct((NP, H), jnp.float32) for _ in range(3)],
)


def _relu_sum(agg, din, bs):
    acc = bs[0][...] + bs[1][...] + bs[2][...]
    acc = jnp.broadcast_to(acc, (BN, H)).astype(jnp.float32)
    for r in range(3):
        acc = acc + agg[r][...] * _csrc(din[r][...])
    return jnp.maximum(acc, 0.0)


def _mid_body(*refs):
    agg = refs[0:3]
    din = refs[3:6]
    dout = refs[6:9]
    ws = refs[9:12]
    bs = refs[12:15]
    out_refs = refs[15:18]
    h1 = _relu_sum(agg, din, bs)
    for r in range(3):
        out_refs[r][...] = jnp.dot(h1 * _csrc(dout[r][...]), ws[r][...],
                                   preferred_element_type=jnp.float32)


_mid = pl.pallas_call(
    _mid_body,
    grid=(GP,),
    in_specs=[pl.BlockSpec((BN, H), _clamped) for _ in range(3)]
    + [pl.BlockSpec((BN, DEGW), _clamped) for _ in range(6)]
    + [pl.BlockSpec((D, H), lambda i: (0, 0)) for _ in range(3)]
    + [pl.BlockSpec((1, H), lambda i: (0, 0)) for _ in range(3)],
    out_specs=[pl.BlockSpec((BN, H), lambda i: (i, 0)) for _ in range(3)],
    out_shape=[jax.ShapeDtypeStruct((NP, H), jnp.float32) for _ in range(3)],
)


def _post_body(*refs):
    agg = refs[0:3]
    din = refs[3:6]
    bs = refs[6:9]
    refs[9][...] = _relu_sum(agg, din, bs)


_post = pl.pallas_call(
    _post_body,
    grid=(GN,),
    in_specs=[pl.BlockSpec((BN, H), lambda i: (i, 0)) for _ in range(3)]
    + [pl.BlockSpec((BN, DEGW), lambda i: (i, 0)) for _ in range(3)]
    + [pl.BlockSpec((1, H), lambda i: (0, 0)) for _ in range(3)],
    out_specs=pl.BlockSpec((BN, H), lambda i: (i, 0)),
    out_shape=jax.ShapeDtypeStruct((N, H), jnp.float32),
)


@jax.jit
def kernel(x, edge_index_rel0, edge_index_rel1, edge_index_rel2,
           W0_rel0, W0_rel1, W0_rel2, b0_rel0, b0_rel1, b0_rel2,
           W1_rel0, W1_rel1, W1_rel2, b1_rel0, b1_rel1, b1_rel2):
    pad = jnp.full((EPAD - E,), N, dtype=jnp.int32)
    edges = []
    for ei in (edge_index_rel0, edge_index_rel1, edge_index_rel2):
        edges.append(jnp.concatenate([ei[0], pad]))
        edges.append(jnp.concatenate([ei[1], pad]))

    ones_h = jnp.ones((C, DEGW), jnp.float32)
    zeros_deg = jnp.zeros((DROWCH, DEGW), jnp.float32)

    degs = _deg_kernel(*edges, ones_h, zeros_deg)
    dego = degs[0:3]
    degi = degs[3:6]

    b0s = [b0_rel0.reshape(1, H), b0_rel1.reshape(1, H), b0_rel2.reshape(1, H)]
    b1s = [b1_rel0.reshape(1, H), b1_rel1.reshape(1, H), b1_rel2.reshape(1, H)]

    tables0 = _prep(x, *dego, W0_rel0, W0_rel1, W0_rel2)
    aggs0 = _agg_kernel(*[t.reshape(NF, F) for t in tables0], *edges)
    tables1 = _mid(*[a.reshape(NP, H) for a in aggs0], *degi, *dego,
                   W1_rel0, W1_rel1, W1_rel2, *b0s)
    aggs1 = _agg_kernel(*[t.reshape(NF, F) for t in tables1], *edges)
    return _post(*[a.reshape(NP, H) for a in aggs1], *degi, *b1s)


# speedup vs baseline: 2.8922x; 1.0179x over previous
import functools
import jax
import jax.numpy as jnp
from jax import lax
from jax.experimental import pallas as pl
from jax.experimental.pallas import tpu as pltpu, tpu_sc as plsc

N = 50000
E = 200000
D = 128
H = 128
C = 128
NS = 16
Q = 100
EPAD = NS * Q * C
NSLOT = 8
GLA = 4
WSLOT = 4
TCH = 25
NP = NS * TCH * C
NF = 8 * NP
F = 16
NBLK = H // F
BPC = NBLK // 2
ZCH = 1024
DEGW = 8
DTCH = 8
DROWCH = NP // (NS * DTCH)

_mesh = plsc.VectorSubcoreMesh(core_axis_name="c", subcore_axis_name="s")
_sc_params = pltpu.CompilerParams(use_tc_tiling_on_sc=False)


@functools.partial(
    pl.kernel,
    out_type=[jax.ShapeDtypeStruct((N, DEGW), jnp.float32) for _ in range(6)],
    mesh=_mesh,
    scratch_types=[
        pltpu.VMEM((Q * C,), jnp.int32),
        pltpu.VMEM((C,), jnp.int32),
        pltpu.VMEM((C,), jnp.int32),
        pltpu.VMEM((DROWCH, DEGW), jnp.float32),
        pltpu.VMEM((DROWCH, DEGW), jnp.float32),
        pltpu.VMEM((C, DEGW), jnp.float32),
        pltpu.VMEM_SHARED((NP, DEGW), jnp.float32),
        pltpu.SemaphoreType.DMA,
        pltpu.SemaphoreType.DMA,
    ],
    compiler_params=_sc_params,
)
def _deg_kernel(src0, dst0, src1, dst1, src2, dst2, ones_h, zeros_h,
                o0, o1, o2, i0, i1, i2,
                idx_blk, ix0, ix1, zero_v, wbd_v, ones_v, acc, sem0, sem1):
    c = lax.axis_index("c")
    s = lax.axis_index("s")
    srcs = [src0, src1, src2]
    dsts = [dst0, dst1, dst2]
    outs_o = [o0, o1, o2]
    outs_i = [i0, i1, i2]
    ixs = [ix0, ix1]
    sems = [sem0, sem1]

    pltpu.sync_copy(ones_h, ones_v)
    pltpu.sync_copy(zeros_h, zero_v)

    for side in range(2):
        @pl.when(c == side)
        def _():
            edges = dsts if side else srcs
            outs = outs_i if side else outs_o
            for r in range(3):
                def zero_chunk(i, _):
                    pltpu.sync_copy(zero_v, acc.at[pl.ds((s * DTCH + i) * DROWCH, DROWCH)])
                    return 0
                lax.fori_loop(0, DTCH, zero_chunk, 0)
                plsc.subcore_barrier()

                pltpu.sync_copy(edges[r].at[pl.ds(s * Q * C, Q * C)], idx_blk)

                def fill_ix(j, k):
                    for g in range(C // 16):
                        ixs[k][pl.ds(g * 16, 16)] = idx_blk[pl.ds(j * C + g * 16, 16)]

                def fire(k):
                    pltpu.async_copy(ones_v, acc.at[ixs[k]], sems[k], add=True)

                for k in range(2):
                    fill_ix(k, k)
                    fire(k)

                def edge_pair(i, _):
                    j = i * 2
                    for k in range(2):
                        pltpu.make_async_copy(ones_v, acc.at[ixs[k]], sems[k]).wait()
                        jn = j + k + 2
                        @pl.when(jn < Q)
                        def _():
                            fill_ix(jn, k)
                            fire(k)
                    return 0
                lax.fori_loop(0, Q // 2, edge_pair, 0)
                plsc.subcore_barrier()

                def wb_chunk(i, _):
                    base = (s * DTCH + i) * DROWCH
                    @pl.when(base < N)
                    def _():
                        pltpu.sync_copy(acc.at[pl.ds(base, DROWCH)], wbd_v)
                        pltpu.sync_copy(wbd_v, outs[r].at[pl.ds(base, DROWCH)])
                    return 0
                lax.fori_loop(0, DTCH, wb_chunk, 0)
                plsc.subcore_barrier()


@functools.partial(
    pl.kernel,
    out_type=[jax.ShapeDtypeStruct((NF, F), jnp.float32) for _ in range(3)],
    mesh=_mesh,
    scratch_types=[
        pltpu.VMEM((Q * C,), jnp.int32),
        pltpu.VMEM((Q * C,), jnp.int32),
        [pltpu.VMEM((C,), jnp.int32) for _ in range(NSLOT)],
        [pltpu.VMEM((C,), jnp.int32) for _ in range(NSLOT)],
        [pltpu.VMEM((C, F), jnp.float32) for _ in range(NSLOT)],
        [pltpu.VMEM((C,), jnp.int32) for _ in range(WSLOT)],
        [pltpu.VMEM((C, F), jnp.float32) for _ in range(WSLOT)],
        pltpu.VMEM((ZCH, F), jnp.float32),
        pltpu.VMEM_SHARED((NP, F), jnp.float32),
        [pltpu.SemaphoreType.DMA for _ in range(NSLOT)],
        [pltpu.SemaphoreType.DMA for _ in range(NSLOT)],
        [pltpu.SemaphoreType.DMA for _ in range(WSLOT)],
    ],
    compiler_params=_sc_params,
)
def _agg_kernel(t0, t1, t2, src0, dst0, src1, dst1, src2, dst2,
                a0, a1, a2,
                src_blk, dst_blk, gi_v, di_v, msg_v, wi_v, wb_v, z_v, acc,
                gsem, ssem, wsem):
    c = lax.axis_index("c")
    s = lax.axis_index("s")
    tables = [t0, t1, t2]
    srcs = [src0, src1, src2]
    dsts = [dst0, dst1, dst2]
    outs = [a0, a1, a2]
    iota = lax.iota(jnp.int32, 16)

    def zz(i, _):
        z_v[i, :] = jnp.zeros((F,), jnp.float32)
        return 0
    lax.fori_loop(0, ZCH, zz, 0)

    base_e = s * Q * C
    base_n = s * TCH * C

    for r in range(3):
        pltpu.sync_copy(srcs[r].at[pl.ds(base_e, Q * C)], src_blk)
        pltpu.sync_copy(dsts[r].at[pl.ds(base_e, Q * C)], dst_blk)

        def block_body(b, _):
            @pl.when(c == b // BPC)
            def _():
                for q in range(TCH * C // ZCH):
                    pltpu.sync_copy(z_v, acc.at[pl.ds(base_n + q * ZCH, ZCH)])
                rem = TCH * C - (TCH * C // ZCH) * ZCH
                if rem:
                    pltpu.sync_copy(z_v.at[pl.ds(0, rem)],
                                    acc.at[pl.ds(base_n + (TCH * C // ZCH) * ZCH, rem)])
                plsc.subcore_barrier()

                def prep(j, k):
                    for g in range(C // 16):
                        sv = src_blk[pl.ds(j * C + g * 16, 16)]
                        gi_v[k][pl.ds(g * 16, 16)] = sv * 8 + b
                        di_v[k][pl.ds(g * 16, 16)] = dst_blk[pl.ds(j * C + g * 16, 16)]

                def g_fire(k):
                    pltpu.async_copy(tables[r].at[gi_v[k]], msg_v[k], gsem[k])

                def g_wait(k):
                    pltpu.make_async_copy(tables[r].at[gi_v[k]], msg_v[k], gsem[k]).wait()

                def s_fire(k):
                    pltpu.async_copy(msg_v[k], acc.at[di_v[k]], ssem[k], add=True)

                def s_wait(k):
                    pltpu.make_async_copy(msg_v[k], acc.at[di_v[k]], ssem[k]).wait()

                for k in range(GLA):
                    prep(k, k)
                    g_fire(k)

                def edge_group(i, _):
                    j = i * NSLOT
                    for k in range(NSLOT):
                        m = j + k
                        @pl.when(m < Q)
                        def _():
                            g_wait(k)
                            s_fire(k)
                        mf = m + GLA
                        kf = (k + GLA) % NSLOT
                        @pl.when((mf >= NSLOT) & (mf < Q))
                        def _():
                            s_wait(kf)
                        @pl.when((mf >= GLA) & (mf < Q))
                        def _():
                            prep(mf, kf)
                            g_fire(kf)
                    return 0
                lax.fori_loop(0, pl.cdiv(Q, NSLOT), edge_group, 0)
                for k in range(NSLOT):
                    s_wait(k)
                plsc.subcore_barrier()

                def wb_prep(m, k):
                    pltpu.sync_copy(acc.at[pl.ds(base_n + m * C, C)], wb_v[k])
                    for g in range(C // 16):
                        wi_v[k][pl.ds(g * 16, 16)] = (base_n + m * C + g * 16 + iota) * 8 + b

                def wb_fire(k):
                    pltpu.async_copy(wb_v[k], outs[r].at[wi_v[k]], wsem[k])

                def wb_wait(k):
                    pltpu.make_async_copy(wb_v[k], outs[r].at[wi_v[k]], wsem[k]).wait()

                def wb_group(i, _):
                    for k in range(WSLOT):
                        m = i * WSLOT + k
                        @pl.when((m >= WSLOT) & (m < TCH))
                        def _():
                            wb_wait(k)
                        @pl.when(m < TCH)
                        def _():
                            wb_prep(m, k)
                            wb_fire(k)
                    return 0
                lax.fori_loop(0, pl.cdiv(TCH, WSLOT), wb_group, 0)
                for k in range(WSLOT):
                    wb_wait(k)
                plsc.subcore_barrier()
            return 0

        lax.fori_loop(0, NBLK, block_body, 0)


BN = 1000
GN = N // BN
GP = (NP + BN - 1) // BN


def _clamped(i):
    return (jnp.minimum(i, GN - 1), 0)


def _csrc(deg_blk):
    return lax.rsqrt(jnp.maximum(deg_blk[:, :1], 1.0))


def _prep_body(x_ref, d0_ref, d1_ref, d2_ref, w0_ref, w1_ref, w2_ref,
               o0_ref, o1_ref, o2_ref):
    h = x_ref[...]
    degs = [d0_ref, d1_ref, d2_ref]
    ws = [w0_ref, w1_ref, w2_ref]
    for r, o_ref in enumerate([o0_ref, o1_ref, o2_ref]):
        o_ref[...] = jnp.dot(h * _csrc(degs[r][...]), ws[r][...],
                             preferred_element_type=jnp.float32)


_prep = pl.pallas_call(
    _prep_body,
    grid=(GP,),
    in_specs=[pl.BlockSpec((BN, D), _clamped)]
    + [pl.BlockSpec((BN, DEGW), _clamped) for _ in range(3)]
    + [pl.BlockSpec((D, H), lambda i: (0, 0)) for _ in range(3)],
    out_specs=[pl.BlockSpec((BN, H), lambda i: (i, 0)) for _ in range(3)],
    out_shape=[jax.ShapeDtypeStruct((NP, H), jnp.float32) for _ in range(3)],
)


def _relu_sum(agg, din, bs):
    acc = bs[0][...] + bs[1][...] + bs[2][...]
    acc = jnp.broadcast_to(acc, (BN, H)).astype(jnp.float32)
    for r in range(3):
        acc = acc + agg[r][...] * _csrc(din[r][...])
    return jnp.maximum(acc, 0.0)


def _mid_body(*refs):
    agg = refs[0:3]
    din = refs[3:6]
    dout = refs[6:9]
    ws = refs[9:12]
    bs = refs[12:15]
    out_refs = refs[15:18]
    h1 = _relu_sum(agg, din, bs)
    for r in range(3):
        out_refs[r][...] = jnp.dot(h1 * _csrc(dout[r][...]), ws[r][...],
                                   preferred_element_type=jnp.float32)


_mid = pl.pallas_call(
    _mid_body,
    grid=(GP,),
    in_specs=[pl.BlockSpec((BN, H), _clamped) for _ in range(3)]
    + [pl.BlockSpec((BN, DEGW), _clamped) for _ in range(6)]
    + [pl.BlockSpec((D, H), lambda i: (0, 0)) for _ in range(3)]
    + [pl.BlockSpec((1, H), lambda i: (0, 0)) for _ in range(3)],
    out_specs=[pl.BlockSpec((BN, H), lambda i: (i, 0)) for _ in range(3)],
    out_shape=[jax.ShapeDtypeStruct((NP, H), jnp.float32) for _ in range(3)],
)


def _post_body(*refs):
    agg = refs[0:3]
    din = refs[3:6]
    bs = refs[6:9]
    refs[9][...] = _relu_sum(agg, din, bs)


_post = pl.pallas_call(
    _post_body,
    grid=(GN,),
    in_specs=[pl.BlockSpec((BN, H), lambda i: (i, 0)) for _ in range(3)]
    + [pl.BlockSpec((BN, DEGW), lambda i: (i, 0)) for _ in range(3)]
    + [pl.BlockSpec((1, H), lambda i: (0, 0)) for _ in range(3)],
    out_specs=pl.BlockSpec((BN, H), lambda i: (i, 0)),
    out_shape=jax.ShapeDtypeStruct((N, H), jnp.float32),
)


@jax.jit
def kernel(x, edge_index_rel0, edge_index_rel1, edge_index_rel2,
           W0_rel0, W0_rel1, W0_rel2, b0_rel0, b0_rel1, b0_rel2,
           W1_rel0, W1_rel1, W1_rel2, b1_rel0, b1_rel1, b1_rel2):
    pad = jnp.full((EPAD - E,), N, dtype=jnp.int32)
    edges = []
    for ei in (edge_index_rel0, edge_index_rel1, edge_index_rel2):
        edges.append(jnp.concatenate([ei[0], pad]))
        edges.append(jnp.concatenate([ei[1], pad]))

    ones_h = jnp.ones((C, DEGW), jnp.float32)
    zeros_deg = jnp.zeros((DROWCH, DEGW), jnp.float32)

    degs = _deg_kernel(*edges, ones_h, zeros_deg)
    dego = degs[0:3]
    degi = degs[3:6]

    b0s = [b0_rel0.reshape(1, H), b0_rel1.reshape(1, H), b0_rel2.reshape(1, H)]
    b1s = [b1_rel0.reshape(1, H), b1_rel1.reshape(1, H), b1_rel2.reshape(1, H)]

    tables0 = _prep(x, *dego, W0_rel0, W0_rel1, W0_rel2)
    aggs0 = _agg_kernel(*[t.reshape(NF, F) for t in tables0], *edges)
    tables1 = _mid(*[a.reshape(NP, H) for a in aggs0], *degi, *dego,
                   W1_rel0, W1_rel1, W1_rel2, *b0s)
    aggs1 = _agg_kernel(*[t.reshape(NF, F) for t in tables1], *edges)
    return _post(*[a.reshape(NP, H) for a in aggs1], *degi, *b1s)
